# Initial kernel scaffold; baseline (speedup 1.0000x reference)
import jax
import jax.numpy as jnp
from jax.experimental import pallas as pl


def kernel(x, points, bounds):
    raise NotImplementedError("write your pallas kernel here")


# speedup vs baseline: 423.1858x; 423.1858x over previous
import functools

import jax
import jax.numpy as jnp
from jax import lax
from jax.experimental import pallas as pl
from jax.experimental.pallas import tpu as pltpu
from jax.experimental.pallas import tpu_sc as plsc

_NC = 2
_NS = 16
_L = 16


@functools.partial(jax.jit, static_argnums=(3, 4))
def _pwlu_sc(x2, pts_flat, par, rows, plane):
    nw = _NC * _NS
    rows_per_w = rows // nw
    nvec = plane // _L
    nchan = par.shape[1]

    mesh = plsc.VectorSubcoreMesh(core_axis_name="c", subcore_axis_name="s")

    @functools.partial(
        pl.kernel,
        out_type=jax.ShapeDtypeStruct((rows, plane), jnp.float32),
        mesh=mesh,
        scratch_types=[
            pltpu.VMEM((plane,), jnp.float32),
            pltpu.VMEM(pts_flat.shape, jnp.float32),
            pltpu.VMEM(par.shape, jnp.float32),
        ],
        compiler_params=pltpu.CompilerParams(needs_layout_passes=False),
    )
    def body(x_hbm, pts_hbm, par_hbm, out_hbm, buf, pts_v, par_v):
        wid = lax.axis_index("s") * _NC + lax.axis_index("c")
        pltpu.sync_copy(pts_hbm, pts_v)
        pltpu.sync_copy(par_hbm, par_v)

        def row_body(i, carry):
            rid = wid * rows_per_w + i
            c = lax.rem(rid, nchan)
            a = par_v[0, c]
            b = par_v[1, c]
            lo = par_v[2, c]
            hi = par_v[3, c]
            pltpu.sync_copy(x_hbm.at[rid], buf)

            def vec_body(j, carry2):
                sl = pl.ds(j * _L, _L)
                v = buf[sl]
                t = v * a + b
                tc = jnp.minimum(jnp.maximum(t, lo), hi)
                ri = tc.astype(jnp.int32)
                d = t - ri.astype(jnp.float32)
                left = plsc.load_gather(pts_v, [ri])
                right = plsc.load_gather(pts_v, [ri + 1])
                buf[sl] = left + d * (right - left)
                return carry2

            lax.fori_loop(0, nvec, vec_body, 0)
            pltpu.sync_copy(buf, out_hbm.at[rid])
            return carry

        lax.fori_loop(0, rows_per_w, row_body, 0)

    return body(x2, pts_flat, par)


def kernel(x, points, bounds):
    b, c, h, w = x.shape
    n_regions = points.shape[1] - 1
    rows = b * c
    plane = h * w

    x2 = x.reshape(rows, plane)
    pts_pad = jnp.zeros((c, 16), jnp.float32).at[:, : points.shape[1]].set(points)
    pts_flat = pts_pad.reshape(-1)

    lo = bounds[:, 0]
    hi = bounds[:, 1]
    a = n_regions / (hi - lo)
    base = 16.0 * jnp.arange(c, dtype=jnp.float32)
    par = jnp.stack(
        [a, -lo * a + base, base, base + jnp.float32(0.999 * n_regions)]
    )
    par = jnp.broadcast_to(par[:, :, None], (4, c, _L))

    out = _pwlu_sc(x2, pts_flat, par, rows, plane)
    return out.reshape(b, c, h, w)


# speedup vs baseline: 1437.7607x; 3.3975x over previous
import functools

import jax
import jax.numpy as jnp
from jax import lax
from jax.experimental import pallas as pl
from jax.experimental.pallas import tpu as pltpu
from jax.experimental.pallas import tpu_sc as plsc

_NC = 2
_NS = 16
_L = 16
_NBUF = 4
_SPLIT = 4


@functools.partial(jax.jit, static_argnums=(4, 5, 6))
def _pwlu_sc(x2, lpts, spts, par, rows, plane, nchan):
    nw = _NC * _NS
    rows_per_w = rows // nw
    nvec = plane // _L

    mesh = plsc.VectorSubcoreMesh(core_axis_name="c", subcore_axis_name="s")

    @functools.partial(
        pl.kernel,
        out_type=jax.ShapeDtypeStruct((rows, plane), jnp.float32),
        mesh=mesh,
        scratch_types=[pltpu.VMEM((plane,), jnp.float32)] * _NBUF
        + [
            pltpu.VMEM(lpts.shape, jnp.float32),
            pltpu.VMEM(spts.shape, jnp.float32),
            pltpu.VMEM(par.shape, jnp.float32),
        ]
        + [pltpu.SemaphoreType.DMA] * (2 * _NBUF),
        compiler_params=pltpu.CompilerParams(needs_layout_passes=False),
    )
    def body(x_hbm, l_hbm, s_hbm, par_hbm, out_hbm, *rest):
        bufs = rest[:_NBUF]
        lv, sv, parv = rest[_NBUF : _NBUF + 3]
        slds = rest[_NBUF + 3 : 2 * _NBUF + 3]
        ssts = rest[2 * _NBUF + 3 :]
        wid = lax.axis_index("s") * _NC + lax.axis_index("c")
        base_row = wid * rows_per_w
        pltpu.sync_copy(l_hbm, lv)
        pltpu.sync_copy(s_hbm, sv)
        pltpu.sync_copy(par_hbm, parv)

        def start_load(k, b):
            pltpu.async_copy(x_hbm.at[base_row + k], bufs[b], slds[b])

        def wait_load(b):
            pltpu.make_async_copy(x_hbm.at[0], bufs[b], slds[b]).wait()

        def start_store(k, b):
            pltpu.async_copy(bufs[b], out_hbm.at[base_row + k], ssts[b])

        def wait_store(b):
            pltpu.make_async_copy(bufs[b], out_hbm.at[0], ssts[b]).wait()

        for b in range(_NBUF - 1):
            start_load(b, b)

        def outer(i, carry):
            for b in range(_NBUF):
                k = _NBUF * i + b
                c = lax.rem(lax.div(base_row + k, _SPLIT), nchan)
                wait_load(b)
                a = parv[0, c]
                off = parv[1, c]
                lo = parv[2, c]
                hi = parv[3, c]
                bufb = bufs[b]

                @plsc.parallel_loop(0, nvec, unroll=8)
                def _(j):
                    sl = pl.ds(j * _L, _L)
                    v = bufb[sl]
                    t = v * a + off
                    tc = jnp.minimum(jnp.maximum(t, lo), hi)
                    ri = tc.astype(jnp.int32)
                    d = t - ri.astype(jnp.float32)
                    left = plsc.load_gather(lv, [ri])
                    slope = plsc.load_gather(sv, [ri])
                    bufb[sl] = left + d * slope

                start_store(k, b)
                p = (b + _NBUF - 1) % _NBUF

                @pl.when(k + _NBUF - 1 < rows_per_w)
                def _():
                    @pl.when(k > 0)
                    def _():
                        wait_store(p)

                    start_load(k + _NBUF - 1, p)
            return carry

        lax.fori_loop(0, rows_per_w // _NBUF, outer, 0)
        for b in range(_NBUF):
            wait_store(b)

    return body(x2, lpts, spts, par)


def kernel(x, points, bounds):
    b, c, h, w = x.shape
    n_regions = points.shape[1] - 1
    rows = b * c * _SPLIT
    plane = (h * w) // _SPLIT

    x2 = x.reshape(rows, plane)
    lpts = jnp.zeros((c, 16), jnp.float32).at[:, : points.shape[1]].set(points)
    spts = (
        jnp.zeros((c, 16), jnp.float32)
        .at[:, :n_regions]
        .set(points[:, 1:] - points[:, :-1])
    )
    lo = bounds[:, 0]
    hi = bounds[:, 1]
    a = n_regions / (hi - lo)
    base = 16.0 * jnp.arange(c, dtype=jnp.float32)
    par = jnp.stack(
        [a, -lo * a + base, base, base + jnp.float32(0.999 * n_regions)]
    )
    par = jnp.broadcast_to(par[:, :, None], (4, c, _L))

    out = _pwlu_sc(x2, lpts.reshape(-1), spts.reshape(-1), par, rows, plane, c)
    return out.reshape(b, c, h, w)
